# Initial kernel scaffold; baseline (speedup 1.0000x reference)
import jax
import jax.numpy as jnp
from jax.experimental import pallas as pl


def kernel(temporal_type_ids, embedding_table):
    raise NotImplementedError("write your pallas kernel here")


# speedup vs baseline: 5.2891x; 5.2891x over previous
import functools

import jax
import jax.numpy as jnp
from jax import lax
from jax.experimental import pallas as pl
from jax.experimental.pallas import tpu as pltpu
from jax.experimental.pallas import tpu_sc as plsc

NUM_TYPES = 1000
DIM = 64
BATCH = 16384
HIST = 50
TOTAL = BATCH * HIST
NC, NS = 2, 16
NW = NC * NS
ROWS_PER_W = TOTAL // NW
CHUNK = 1024
NSTEPS = ROWS_PER_W // CHUNK


def _body(ids_hbm, table_hbm, out_hbm, idx_v, rows_v, sem):
    wid = lax.axis_index("s") * NC + lax.axis_index("c")
    base_w = wid * ROWS_PER_W

    def step(t, _):
        base = base_w + t * CHUNK
        pltpu.sync_copy(ids_hbm.at[pl.ds(base, CHUNK)], idx_v)
        pltpu.async_copy(table_hbm.at[idx_v], rows_v, sem).wait()
        pltpu.sync_copy(rows_v, out_hbm.at[pl.ds(base, CHUNK)])
        return 0

    lax.fori_loop(0, NSTEPS, step, 0)


@jax.jit
def _lookup(ids_flat, table):
    mesh = plsc.VectorSubcoreMesh(core_axis_name="c", subcore_axis_name="s")
    return pl.kernel(
        _body,
        out_type=jax.ShapeDtypeStruct((TOTAL, DIM), jnp.float32),
        mesh=mesh,
        scratch_types=[
            pltpu.VMEM((CHUNK,), jnp.int32),
            pltpu.VMEM((CHUNK, DIM), jnp.float32),
            pltpu.SemaphoreType.DMA,
        ],
        compiler_params=pltpu.CompilerParams(use_tc_tiling_on_sc=False),
    )(ids_flat, table)


def kernel(temporal_type_ids, embedding_table):
    ids_flat = temporal_type_ids.reshape(TOTAL).astype(jnp.int32)
    out = _lookup(ids_flat, embedding_table)
    return out.reshape(BATCH, HIST, DIM)


# speedup vs baseline: 6.6310x; 1.2537x over previous
import functools

import jax
import jax.numpy as jnp
from jax import lax
from jax.experimental import pallas as pl
from jax.experimental.pallas import tpu as pltpu
from jax.experimental.pallas import tpu_sc as plsc

NUM_TYPES = 1000
DIM = 64
BATCH = 16384
HIST = 50
TOTAL = BATCH * HIST
NC, NS = 2, 16
NW = NC * NS
ROWS_PER_W = TOTAL // NW
CHUNK = 1024
NSTEPS = ROWS_PER_W // CHUNK


def _body(ids_hbm, table_hbm, out_hbm, table_sh, idx_v, rows_v, sem):
    sid = lax.axis_index("s")
    wid = sid * NC + lax.axis_index("c")
    base_w = wid * ROWS_PER_W

    @pl.when(sid == 0)
    def _():
        pltpu.sync_copy(table_hbm, table_sh)

    plsc.subcore_barrier()

    def step(t, _):
        base = base_w + t * CHUNK
        pltpu.sync_copy(ids_hbm.at[pl.ds(base, CHUNK)], idx_v)
        pltpu.async_copy(table_sh.at[idx_v], rows_v, sem).wait()
        pltpu.sync_copy(rows_v, out_hbm.at[pl.ds(base, CHUNK)])
        return 0

    lax.fori_loop(0, NSTEPS, step, 0)


@jax.jit
def _lookup(ids_flat, table):
    mesh = plsc.VectorSubcoreMesh(core_axis_name="c", subcore_axis_name="s")
    return pl.kernel(
        _body,
        out_type=jax.ShapeDtypeStruct((TOTAL, DIM), jnp.float32),
        mesh=mesh,
        scratch_types=[
            pltpu.VMEM_SHARED((NUM_TYPES, DIM), jnp.float32),
            pltpu.VMEM((CHUNK,), jnp.int32),
            pltpu.VMEM((CHUNK, DIM), jnp.float32),
            pltpu.SemaphoreType.DMA,
        ],
        compiler_params=pltpu.CompilerParams(use_tc_tiling_on_sc=False),
    )(ids_flat, table)


def kernel(temporal_type_ids, embedding_table):
    ids_flat = temporal_type_ids.reshape(TOTAL).astype(jnp.int32)
    out = _lookup(ids_flat, embedding_table)
    return out.reshape(BATCH, HIST, DIM)


# speedup vs baseline: 6.8297x; 1.0300x over previous
import functools

import jax
import jax.numpy as jnp
from jax import lax
from jax.experimental import pallas as pl
from jax.experimental.pallas import tpu as pltpu
from jax.experimental.pallas import tpu_sc as plsc

NUM_TYPES = 1000
DIM = 64
BATCH = 16384
HIST = 50
TOTAL = BATCH * HIST
NC, NS = 2, 16
NW = NC * NS
ROWS_PER_W = TOTAL // NW
CHUNK = 512
NSTEPS = ROWS_PER_W // CHUNK


def _body(ids_hbm, table_hbm, out_hbm, table_sh, idx_v, rows_v,
          gsem0, gsem1, osem0, osem1):
    sid = lax.axis_index("s")
    wid = sid * NC + lax.axis_index("c")
    base_w = wid * ROWS_PER_W
    gsem = (gsem0, gsem1)
    osem = (osem0, osem1)

    @pl.when(sid == 0)
    def _():
        pltpu.sync_copy(table_hbm, table_sh)

    pltpu.sync_copy(ids_hbm.at[wid], idx_v)
    plsc.subcore_barrier()

    def start_gather(t, b):
        pltpu.async_copy(table_sh.at[idx_v.at[t]], rows_v.at[b], gsem[b])

    def wait_gather(b):
        pltpu.make_async_copy(table_sh.at[idx_v.at[0]], rows_v.at[b],
                              gsem[b]).wait()

    def start_out(t, b):
        pltpu.async_copy(rows_v.at[b], out_hbm.at[pl.ds(base_w + t * CHUNK,
                                                        CHUNK)], osem[b])

    def wait_out(b):
        pltpu.make_async_copy(rows_v.at[b], out_hbm.at[pl.ds(base_w, CHUNK)],
                              osem[b]).wait()

    start_gather(0, 0)
    start_gather(1, 1)

    def step(g, _):
        t0 = 2 * g
        wait_gather(0)
        start_out(t0, 0)
        wait_gather(1)
        start_out(t0 + 1, 1)
        wait_out(0)
        start_gather(t0 + 2, 0)
        wait_out(1)
        start_gather(t0 + 3, 1)
        return 0

    lax.fori_loop(0, NSTEPS // 2 - 1, step, 0)

    wait_gather(0)
    start_out(NSTEPS - 2, 0)
    wait_gather(1)
    start_out(NSTEPS - 1, 1)
    wait_out(0)
    wait_out(1)


@jax.jit
def _lookup(ids3, table):
    mesh = plsc.VectorSubcoreMesh(core_axis_name="c", subcore_axis_name="s")
    return pl.kernel(
        _body,
        out_type=jax.ShapeDtypeStruct((TOTAL, DIM), jnp.float32),
        mesh=mesh,
        scratch_types=[
            pltpu.VMEM_SHARED((NUM_TYPES, DIM), jnp.float32),
            pltpu.VMEM((NSTEPS, CHUNK), jnp.int32),
            pltpu.VMEM((2, CHUNK, DIM), jnp.float32),
            pltpu.SemaphoreType.DMA,
            pltpu.SemaphoreType.DMA,
            pltpu.SemaphoreType.DMA,
            pltpu.SemaphoreType.DMA,
        ],
        compiler_params=pltpu.CompilerParams(use_tc_tiling_on_sc=False),
    )(ids3, table)


def kernel(temporal_type_ids, embedding_table):
    ids3 = temporal_type_ids.reshape(NW, NSTEPS, CHUNK).astype(jnp.int32)
    out = _lookup(ids3, embedding_table)
    return out.reshape(BATCH, HIST, DIM)


# speedup vs baseline: 9.0602x; 1.3266x over previous
import functools

import jax
import jax.numpy as jnp
from jax import lax
from jax.experimental import pallas as pl
from jax.experimental.pallas import tpu as pltpu
from jax.experimental.pallas import tpu_sc as plsc

NUM_TYPES = 1000
DIM = 64
BATCH = 16384
HIST = 50
NC, NS = 2, 16
NW = NC * NS
B_PER_W = BATCH // NW
HB = 256
L = 16


def _body(ids_hbm, tab_hbm, out_hbm, tab_v, ids_v, slab_v, osem0, osem1):
    wid = lax.axis_index("s") * NC + lax.axis_index("c")
    b_w = wid * B_PER_W
    osem = (osem0, osem1)

    pltpu.sync_copy(tab_hbm, tab_v)

    def ids_row_copy(h, _):
        pltpu.sync_copy(ids_hbm.at[pl.ds(h * BATCH + b_w, B_PER_W)],
                        ids_v.at[pl.ds(h * B_PER_W, B_PER_W)])
        return 0

    lax.fori_loop(0, HIST, ids_row_copy, 0)

    def step(h, _):
        for half in range(2):
            @pl.when(h > 0)
            def _():
                pltpu.make_async_copy(
                    slab_v.at[half],
                    out_hbm.at[pl.ds(0, DIM), pl.ds(b_w, HB)],
                    osem[half]).wait()

            def bb_loop(bb, _):
                off = h * B_PER_W + half * HB + bb * L
                idx = ids_v[pl.ds(off, L)]
                col = bb * L
                for d in range(DIM):
                    v = plsc.load_gather(tab_v, [idx + d * NUM_TYPES])
                    slab_v[half, d, pl.ds(col, L)] = v
                return 0

            lax.fori_loop(0, HB // L, bb_loop, 0)
            pltpu.async_copy(
                slab_v.at[half],
                out_hbm.at[pl.ds(h * DIM, DIM),
                           pl.ds(b_w + half * HB, HB)],
                osem[half])
        return 0

    lax.fori_loop(0, HIST, step, 0)

    for half in range(2):
        pltpu.make_async_copy(
            slab_v.at[half],
            out_hbm.at[pl.ds(0, DIM), pl.ds(b_w, HB)],
            osem[half]).wait()


@jax.jit
def _lookup(ids_t_flat, tab_t_flat):
    mesh = plsc.VectorSubcoreMesh(core_axis_name="c", subcore_axis_name="s")
    return pl.kernel(
        _body,
        out_type=jax.ShapeDtypeStruct((HIST * DIM, BATCH), jnp.float32),
        mesh=mesh,
        scratch_types=[
            pltpu.VMEM((NUM_TYPES * DIM,), jnp.float32),
            pltpu.VMEM((HIST * B_PER_W,), jnp.int32),
            pltpu.VMEM((2, DIM, HB), jnp.float32),
            pltpu.SemaphoreType.DMA,
            pltpu.SemaphoreType.DMA,
        ],
        compiler_params=pltpu.CompilerParams(use_tc_tiling_on_sc=True,
                                             needs_layout_passes=False),
    )(ids_t_flat, tab_t_flat)


def kernel(temporal_type_ids, embedding_table):
    ids_t = temporal_type_ids.T.astype(jnp.int32).reshape(HIST * BATCH)
    tab_t = embedding_table.T.reshape(DIM * NUM_TYPES)
    z = _lookup(ids_t, tab_t)
    return jnp.transpose(z.reshape(HIST, DIM, BATCH), (2, 0, 1))


# speedup vs baseline: 13.7956x; 1.5227x over previous
import functools

import jax
import jax.numpy as jnp
from jax import lax
from jax.experimental import pallas as pl
from jax.experimental.pallas import tpu as pltpu
from jax.experimental.pallas import tpu_sc as plsc

NUM_TYPES = 1000
DIM = 64
BATCH = 16384
HIST = 50
NC, NS = 2, 16
NW = NC * NS
B_PER_W = BATCH // NW
HB = 256
L = 16


def _body(ids_hbm, tab_hbm, out_hbm, tab_v, ids_v, slab_v, osem0, osem1):
    wid = lax.axis_index("s") * NC + lax.axis_index("c")
    b_w = wid * B_PER_W
    osem = (osem0, osem1)

    pltpu.sync_copy(tab_hbm, tab_v)

    def ids_row_copy(h, _):
        pltpu.sync_copy(ids_hbm.at[pl.ds(h * BATCH + b_w, B_PER_W)],
                        ids_v.at[pl.ds(h * B_PER_W, B_PER_W)])
        return 0

    lax.fori_loop(0, HIST, ids_row_copy, 0)

    def step(h, _):
        for half in range(2):
            @pl.when(h > 0)
            def _():
                pltpu.make_async_copy(
                    slab_v.at[half],
                    out_hbm.at[pl.ds(0, DIM), pl.ds(b_w, HB)],
                    osem[half]).wait()

            @plsc.parallel_loop(0, HB // L, unroll=2)
            def bb_loop(bb):
                off = h * B_PER_W + half * HB + bb * L
                idx = ids_v[pl.ds(off, L)]
                col = bb * L
                for d in range(DIM):
                    v = plsc.load_gather(tab_v, [idx + d * NUM_TYPES])
                    slab_v[half, d, pl.ds(col, L)] = v
            pltpu.async_copy(
                slab_v.at[half],
                out_hbm.at[pl.ds(h * DIM, DIM),
                           pl.ds(b_w + half * HB, HB)],
                osem[half])
        return 0

    lax.fori_loop(0, HIST, step, 0)

    for half in range(2):
        pltpu.make_async_copy(
            slab_v.at[half],
            out_hbm.at[pl.ds(0, DIM), pl.ds(b_w, HB)],
            osem[half]).wait()


@jax.jit
def _lookup(ids_t_flat, tab_t_flat):
    mesh = plsc.VectorSubcoreMesh(core_axis_name="c", subcore_axis_name="s")
    return pl.kernel(
        _body,
        out_type=jax.ShapeDtypeStruct((HIST * DIM, BATCH), jnp.float32),
        mesh=mesh,
        scratch_types=[
            pltpu.VMEM((NUM_TYPES * DIM,), jnp.float32),
            pltpu.VMEM((HIST * B_PER_W,), jnp.int32),
            pltpu.VMEM((2, DIM, HB), jnp.float32),
            pltpu.SemaphoreType.DMA,
            pltpu.SemaphoreType.DMA,
        ],
        compiler_params=pltpu.CompilerParams(use_tc_tiling_on_sc=True,
                                             needs_layout_passes=False),
    )(ids_t_flat, tab_t_flat)


def kernel(temporal_type_ids, embedding_table):
    ids_t = temporal_type_ids.T.astype(jnp.int32).reshape(HIST * BATCH)
    tab_t = embedding_table.T.reshape(DIM * NUM_TYPES)
    z = _lookup(ids_t, tab_t)
    return jnp.transpose(z.reshape(HIST, DIM, BATCH), (2, 0, 1))


# speedup vs baseline: 20.9951x; 1.5219x over previous
import functools

import jax
import jax.numpy as jnp
from jax import lax
from jax.experimental import pallas as pl
from jax.experimental.pallas import tpu as pltpu
from jax.experimental.pallas import tpu_sc as plsc

NUM_TYPES = 1000
DIM = 64
BATCH = 16384
HIST = 50
NC, NS = 2, 16
NW = NC * NS
B_PER_W = BATCH // NW
HB = 256
L = 16


def _body(ids_hbm, tab_hbm, out_hbm, tab_v, ids_v, slab_v, osem0, osem1):
    wid = lax.axis_index("s") * NC + lax.axis_index("c")
    b_w = wid * B_PER_W
    osem = (osem0, osem1)

    pltpu.sync_copy(tab_hbm, tab_v)

    def ids_row_copy(h, _):
        pltpu.sync_copy(ids_hbm.at[pl.ds(h * BATCH + b_w, B_PER_W)],
                        ids_v.at[pl.ds(h * B_PER_W, B_PER_W)])
        return 0

    lax.fori_loop(0, HIST, ids_row_copy, 0)

    def step(h, _):
        for half in range(2):
            @pl.when(h > 0)
            def _():
                pltpu.make_async_copy(
                    slab_v.at[half],
                    out_hbm.at[pl.ds(0, DIM), pl.ds(b_w, HB)],
                    osem[half]).wait()

            @plsc.parallel_loop(0, HB // L, unroll=4)
            def bb_loop(bb):
                off = h * B_PER_W + half * HB + bb * L
                idx = ids_v[pl.ds(off, L)]
                col = bb * L
                for d in range(DIM):
                    row = tab_v.at[pl.ds(d * NUM_TYPES, NUM_TYPES)]
                    v = plsc.load_gather(row, [idx])
                    slab_v[half, d, pl.ds(col, L)] = v
            pltpu.async_copy(
                slab_v.at[half],
                out_hbm.at[pl.ds(h * DIM, DIM),
                           pl.ds(b_w + half * HB, HB)],
                osem[half])
        return 0

    lax.fori_loop(0, HIST, step, 0)

    for half in range(2):
        pltpu.make_async_copy(
            slab_v.at[half],
            out_hbm.at[pl.ds(0, DIM), pl.ds(b_w, HB)],
            osem[half]).wait()


@jax.jit
def _lookup(ids_t_flat, tab_t_flat):
    mesh = plsc.VectorSubcoreMesh(core_axis_name="c", subcore_axis_name="s")
    return pl.kernel(
        _body,
        out_type=jax.ShapeDtypeStruct((HIST * DIM, BATCH), jnp.float32),
        mesh=mesh,
        scratch_types=[
            pltpu.VMEM((NUM_TYPES * DIM,), jnp.float32),
            pltpu.VMEM((HIST * B_PER_W,), jnp.int32),
            pltpu.VMEM((2, DIM, HB), jnp.float32),
            pltpu.SemaphoreType.DMA,
            pltpu.SemaphoreType.DMA,
        ],
        compiler_params=pltpu.CompilerParams(use_tc_tiling_on_sc=True,
                                             needs_layout_passes=False),
    )(ids_t_flat, tab_t_flat)


def kernel(temporal_type_ids, embedding_table):
    ids_t = temporal_type_ids.T.astype(jnp.int32).reshape(HIST * BATCH)
    tab_t = embedding_table.T.reshape(DIM * NUM_TYPES)
    z = _lookup(ids_t, tab_t)
    return jnp.transpose(z.reshape(HIST, DIM, BATCH), (2, 0, 1))


# speedup vs baseline: 22.3782x; 1.0659x over previous
import functools

import jax
import jax.numpy as jnp
from jax import lax
from jax.experimental import pallas as pl
from jax.experimental.pallas import tpu as pltpu
from jax.experimental.pallas import tpu_sc as plsc

NUM_TYPES = 1000
DIM = 64
BATCH = 16384
HIST = 50
NC, NS = 2, 16
NW = NC * NS
B_PER_W = BATCH // NW
HB = 256
L = 16


def _body(ids_hbm, tab_hbm, out_hbm, tab_v, ids_v, slab_v, osem0, osem1):
    wid = lax.axis_index("s") * NC + lax.axis_index("c")
    b_w = wid * B_PER_W
    osem = (osem0, osem1)

    pltpu.sync_copy(tab_hbm, tab_v)
    pltpu.sync_copy(ids_hbm.at[:, pl.ds(b_w, B_PER_W)], ids_v)

    def step(h, _):
        for half in range(2):
            @pl.when(h > 0)
            def _():
                pltpu.make_async_copy(
                    slab_v.at[half],
                    out_hbm.at[pl.ds(0, DIM), pl.ds(b_w, HB)],
                    osem[half]).wait()

            @plsc.parallel_loop(0, HB // L, unroll=4)
            def bb_loop(bb):
                idx = ids_v[h, pl.ds(half * HB + bb * L, L)]
                col = bb * L
                for d in range(DIM):
                    row = tab_v.at[pl.ds(d * NUM_TYPES, NUM_TYPES)]
                    v = plsc.load_gather(row, [idx])
                    slab_v[half, d, pl.ds(col, L)] = v
            pltpu.async_copy(
                slab_v.at[half],
                out_hbm.at[pl.ds(h * DIM, DIM),
                           pl.ds(b_w + half * HB, HB)],
                osem[half])
        return 0

    lax.fori_loop(0, HIST, step, 0)

    for half in range(2):
        pltpu.make_async_copy(
            slab_v.at[half],
            out_hbm.at[pl.ds(0, DIM), pl.ds(b_w, HB)],
            osem[half]).wait()


@jax.jit
def _lookup(ids_t, tab_t_flat):
    mesh = plsc.VectorSubcoreMesh(core_axis_name="c", subcore_axis_name="s")
    return pl.kernel(
        _body,
        out_type=jax.ShapeDtypeStruct((HIST * DIM, BATCH), jnp.float32),
        mesh=mesh,
        scratch_types=[
            pltpu.VMEM((NUM_TYPES * DIM,), jnp.float32),
            pltpu.VMEM((HIST, B_PER_W), jnp.int32),
            pltpu.VMEM((2, DIM, HB), jnp.float32),
            pltpu.SemaphoreType.DMA,
            pltpu.SemaphoreType.DMA,
        ],
        compiler_params=pltpu.CompilerParams(use_tc_tiling_on_sc=True,
                                             needs_layout_passes=False),
    )(ids_t, tab_t_flat)


def kernel(temporal_type_ids, embedding_table):
    ids_t = temporal_type_ids.T.astype(jnp.int32)
    tab_t = embedding_table.T.reshape(DIM * NUM_TYPES)
    z = _lookup(ids_t, tab_t)
    return jnp.transpose(z.reshape(HIST, DIM, BATCH), (2, 0, 1))


# speedup vs baseline: 26.1295x; 1.1676x over previous
import functools

import jax
import jax.numpy as jnp
from jax import lax
from jax.experimental import pallas as pl
from jax.experimental.pallas import tpu as pltpu
from jax.experimental.pallas import tpu_sc as plsc

NUM_TYPES = 1000
DIM = 64
BATCH = 16384
HIST = 50
NC, NS = 2, 16
NW = NC * NS
B_PER_W = BATCH // NW
HB = 128
L = 16


def _body(ids_hbm, tab_hbm, out_hbm, tab_v, ids_v, slab_v,
          osem0, osem1, osem2, osem3):
    wid = lax.axis_index("s") * NC + lax.axis_index("c")
    b_w = wid * B_PER_W
    osem = (osem0, osem1, osem2, osem3)

    pltpu.sync_copy(tab_hbm, tab_v)
    pltpu.sync_copy(ids_hbm.at[:, pl.ds(b_w, B_PER_W)], ids_v)

    def step(h, _):
        for half in range(4):
            @pl.when(h > 0)
            def _():
                pltpu.make_async_copy(
                    slab_v.at[half],
                    out_hbm.at[pl.ds(0, DIM), pl.ds(b_w, HB)],
                    osem[half]).wait()

            @plsc.parallel_loop(0, HB // L, unroll=4)
            def bb_loop(bb):
                idx = ids_v[h, pl.ds(half * HB + bb * L, L)]
                col = bb * L
                for d in range(DIM):
                    row = tab_v.at[pl.ds(d * NUM_TYPES, NUM_TYPES)]
                    v = plsc.load_gather(row, [idx])
                    slab_v[half, d, pl.ds(col, L)] = v
            pltpu.async_copy(
                slab_v.at[half],
                out_hbm.at[pl.ds(h * DIM, DIM),
                           pl.ds(b_w + half * HB, HB)],
                osem[half])
        return 0

    lax.fori_loop(0, HIST, step, 0)

    for half in range(4):
        pltpu.make_async_copy(
            slab_v.at[half],
            out_hbm.at[pl.ds(0, DIM), pl.ds(b_w, HB)],
            osem[half]).wait()


@jax.jit
def _lookup(ids_t, tab_t_flat):
    mesh = plsc.VectorSubcoreMesh(core_axis_name="c", subcore_axis_name="s")
    return pl.kernel(
        _body,
        out_type=jax.ShapeDtypeStruct((HIST * DIM, BATCH), jnp.float32),
        mesh=mesh,
        scratch_types=[
            pltpu.VMEM((NUM_TYPES * DIM,), jnp.float32),
            pltpu.VMEM((HIST, B_PER_W), jnp.int32),
            pltpu.VMEM((4, DIM, HB), jnp.float32),
            pltpu.SemaphoreType.DMA,
            pltpu.SemaphoreType.DMA,
            pltpu.SemaphoreType.DMA,
            pltpu.SemaphoreType.DMA,
        ],
        compiler_params=pltpu.CompilerParams(use_tc_tiling_on_sc=True,
                                             needs_layout_passes=False),
    )(ids_t, tab_t_flat)


def kernel(temporal_type_ids, embedding_table):
    ids_t = temporal_type_ids.T.astype(jnp.int32)
    tab_t = embedding_table.T.reshape(DIM * NUM_TYPES)
    z = _lookup(ids_t, tab_t)
    return jnp.transpose(z.reshape(HIST, DIM, BATCH), (2, 0, 1))
